# baseline (speedup 1.0000x reference)
import functools

import jax
import jax.numpy as jnp
from jax import lax
from jax.experimental import pallas as pl
from jax.experimental.pallas import tpu as pltpu
from jax.experimental.pallas import tpu_sc as plsc

DIM = 16
NN = 8
L = 16


def _splat(val):
    return jnp.full((L,), val, jnp.int32)


def _tc_transpose_many(tabs, D, dtype, CB=2048):
    n = tabs[0].shape[1]
    grid = (n + CB - 1) // CB

    def body(*refs):
        k = len(refs) // 2
        for i_ref, o_ref in zip(refs[:k], refs[k:]):
            o_ref[...] = jnp.transpose(i_ref[...], (1, 0))

    return pl.pallas_call(
        body,
        grid=(grid,),
        in_specs=[pl.BlockSpec((D, CB), lambda i: (0, i))] * len(tabs),
        out_specs=[pl.BlockSpec((CB, D), lambda i: (i, 0))] * len(tabs),
        out_shape=[jax.ShapeDtypeStruct((grid * CB, D), dtype)] * len(tabs),
    )(*tabs)


@functools.lru_cache(maxsize=None)
def _build(B, NC, NS):
    NW = NC * NS
    BW = B // NW
    CH = 128
    NCHUNK = BW // CH
    GPC = CH // L

    mesh = plsc.VectorSubcoreMesh(core_axis_name="c", subcore_axis_name="s")

    @functools.partial(
        pl.kernel,
        out_type=jax.ShapeDtypeStruct((B,), jnp.float32),
        mesh=mesh,
        compiler_params=pltpu.CompilerParams(
            needs_layout_passes=False, use_tc_tiling_on_sc=False),
        scratch_types=[
            pltpu.VMEM((BW,), jnp.int32),
            pltpu.VMEM((BW,), jnp.int32),
            pltpu.VMEM((64, DIM), jnp.float32),
            pltpu.VMEM((DIM, DIM), jnp.float32),
            pltpu.VMEM((DIM,), jnp.float32),
            pltpu.VMEM((CH, NN), jnp.int32),
            pltpu.VMEM((CH, NN), jnp.int32),
            pltpu.VMEM((CH, NN), jnp.int32),
            pltpu.VMEM((CH, NN), jnp.int32),
            pltpu.VMEM((CH, DIM), jnp.float32),
            pltpu.VMEM((CH, DIM), jnp.float32),
            pltpu.VMEM((CH * NN,), jnp.int32),
            pltpu.VMEM((CH * NN,), jnp.int32),
            pltpu.VMEM((CH * NN,), jnp.int32),
            pltpu.VMEM((CH * NN, DIM), jnp.float32),
            pltpu.VMEM((CH * NN, DIM), jnp.float32),
            pltpu.VMEM((CH * NN, DIM), jnp.float32),
            pltpu.VMEM((BW,), jnp.float32),
            pltpu.SemaphoreType.DMA,
            pltpu.SemaphoreType.DMA,
            pltpu.SemaphoreType.DMA,
            pltpu.SemaphoreType.DMA,
            pltpu.SemaphoreType.DMA,
            pltpu.SemaphoreType.DMA,
        ],
    )
    def klgcn(usr_h, ent_h, rel_h, w_h, b_h, u2i_h, i2u_h, ae_h, ar_h, u_h,
              v_h, out_h,
              u_v, v_v, rel_v, w_v, b_v, nb_u2i, nb_i2u, nb_ae, nb_ar,
              ue_r, io_r, fl_i2u, fl_u2i, fl_ae,
              usr_nb, ent_nb1, ent_nb2, out_v,
              s0, s1, s2, s3, s4, s5):
        wid = lax.axis_index("s") * NC + lax.axis_index("c")
        base = wid * BW
        pltpu.sync_copy(u_h.at[pl.ds(base, BW)], u_v)
        pltpu.sync_copy(v_h.at[pl.ds(base, BW)], v_v)
        pltpu.sync_copy(rel_h, rel_v)
        pltpu.sync_copy(w_h, w_v)
        pltpu.sync_copy(b_h, b_v)
        iota = lax.iota(jnp.int32, L)

        def chunk_body(c, carry):
            off = c * CH
            uc = u_v.at[pl.ds(off, CH)]
            vc = v_v.at[pl.ds(off, CH)]
            d_u2i = pltpu.async_copy(u2i_h.at[uc], nb_u2i, s0)
            d_i2u = pltpu.async_copy(i2u_h.at[vc], nb_i2u, s1)
            d_ae = pltpu.async_copy(ae_h.at[vc], nb_ae, s2)
            d_ar = pltpu.async_copy(ar_h.at[vc], nb_ar, s3)
            d_ue = pltpu.async_copy(usr_h.at[uc], ue_r, s4)
            d_io = pltpu.async_copy(ent_h.at[vc], io_r, s5)
            d_u2i.wait()
            d_i2u.wait()
            d_ae.wait()
            riota = jnp.right_shift(iota, 3)
            ciota = jnp.bitwise_and(iota, 7)

            def flat_body(i, cf):
                ridx = riota + i * 2
                o = i * L
                fl_i2u[pl.ds(o, L)] = plsc.load_gather(nb_i2u, [ridx, ciota])
                fl_u2i[pl.ds(o, L)] = plsc.load_gather(nb_u2i, [ridx, ciota])
                fl_ae[pl.ds(o, L)] = plsc.load_gather(nb_ae, [ridx, ciota])
                return cf

            lax.fori_loop(0, CH * NN // L, flat_body, 0)
            e_usr = pltpu.async_copy(usr_h.at[fl_i2u], usr_nb, s1)
            e_ent1 = pltpu.async_copy(ent_h.at[fl_u2i], ent_nb1, s0)
            e_ent2 = pltpu.async_copy(ent_h.at[fl_ae], ent_nb2, s2)
            d_ar.wait()
            d_ue.wait()
            d_io.wait()
            e_usr.wait()
            e_ent1.wait()
            e_ent2.wait()

            def group_body(g, carry2):
                rows = g * L + iota
                frows = [rows * NN + _splat(j) for j in range(NN)]
                ue = [plsc.load_gather(ue_r, [rows, _splat(d)])
                      for d in range(DIM)]
                s_list = []
                for j in range(NN):
                    relid = plsc.load_gather(nb_ar, [rows, _splat(j)])
                    acc = ue[0] * plsc.load_gather(rel_v, [relid, _splat(0)])
                    for d in range(1, DIM):
                        acc = acc + ue[d] * plsc.load_gather(
                            rel_v, [relid, _splat(d)])
                    s_list.append(acc)
                m = s_list[0]
                for j in range(1, NN):
                    m = jnp.maximum(m, s_list[j])
                e_list = [jnp.exp(sj - m) for sj in s_list]
                tot = e_list[0]
                for j in range(1, NN):
                    tot = tot + e_list[j]
                inv = 1.0 / tot
                p_list = [ej * inv for ej in e_list]
                y = [None] * DIM
                for d in range(DIM):
                    a = p_list[0] * plsc.load_gather(
                        ent_nb2, [frows[0], _splat(d)])
                    for j in range(1, NN):
                        a = a + p_list[j] * plsc.load_gather(
                            ent_nb2, [frows[j], _splat(d)])
                    x_d = a + plsc.load_gather(io_r, [rows, _splat(d)])
                    for dp in range(DIM):
                        w_sc = plsc.load_gather(w_v, [_splat(d), _splat(dp)])
                        term = x_d * w_sc
                        y[dp] = term if y[dp] is None else y[dp] + term
                t_list = []
                for dp in range(DIM):
                    yv = y[dp] + plsc.load_gather(b_v, [_splat(dp)])
                    tt = jnp.exp(jnp.abs(yv) * (-2.0))
                    r = (1.0 - tt) / (1.0 + tt)
                    t_list.append(jnp.where(yv < 0.0, -r, r))
                sc = None
                for d in range(DIM):
                    lu = plsc.load_gather(usr_nb, [frows[0], _splat(d)])
                    for j in range(1, NN):
                        lu = lu + plsc.load_gather(
                            usr_nb, [frows[j], _splat(d)])
                    li = plsc.load_gather(ent_nb1, [frows[0], _splat(d)])
                    for j in range(1, NN):
                        li = li + plsc.load_gather(
                            ent_nb1, [frows[j], _splat(d)])
                    uf = 0.0625 * lu + 0.5 * ue[d]
                    if_ = 0.0625 * li + 0.5 * t_list[d]
                    term = uf * if_
                    sc = term if sc is None else sc + term
                sig = 1.0 / (1.0 + jnp.exp(-sc))
                out_v[pl.ds(off + g * L, L)] = sig
                return carry2

            lax.fori_loop(0, GPC, group_body, 0)
            return carry

        lax.fori_loop(0, NCHUNK, chunk_body, 0)
        pltpu.sync_copy(out_v, out_h.at[pl.ds(base, BW)])

    return klgcn


def kernel(usr, ent, rel, W, b, u2i, i2u, adj_ent, adj_rel, u, v):
    B = u.shape[0]
    usr_rm, ent_rm = _tc_transpose_many([usr.T, ent.T], DIM, jnp.float32)
    u2i_rm, i2u_rm, ae_rm, ar_rm = _tc_transpose_many(
        [u2i.T, i2u.T, adj_ent.T, adj_rel.T], NN, jnp.int32)
    info = plsc.get_sparse_core_info()
    fn = _build(B, info.num_cores, info.num_subcores)
    return fn(usr_rm, ent_rm, rel, W, b, u2i_rm, i2u_rm, ae_rm, ar_rm,
              u.astype(jnp.int32), v.astype(jnp.int32))


# speedup vs baseline: 1.1143x; 1.1143x over previous
import functools

import jax
import jax.numpy as jnp
from jax import lax
from jax.experimental import pallas as pl
from jax.experimental.pallas import tpu as pltpu
from jax.experimental.pallas import tpu_sc as plsc

DIM = 16
NN = 8
L = 16


def _splat(val):
    return jnp.full((L,), val, jnp.int32)


def _tc_transpose_many(tabs, D, dtype, CB=8192):
    n = tabs[0].shape[1]
    grid = (n + CB - 1) // CB

    def body(*refs):
        k = len(refs) // 2
        eye = (lax.broadcasted_iota(jnp.int32, (D, D), 0)
               == lax.broadcasted_iota(jnp.int32, (D, D), 1)
               ).astype(jnp.float32)
        for i_ref, o_ref in zip(refs[:k], refs[k:]):
            x = i_ref[...]
            if dtype == jnp.int32:
                o_ref[...] = jnp.transpose(x, (1, 0))
            else:
                t = lax.dot_general(x, eye, (((0,), (0,)), ((), ())),
                                    preferred_element_type=jnp.float32)
                o_ref[...] = t

    return pl.pallas_call(
        body,
        grid=(grid,),
        in_specs=[pl.BlockSpec((D, CB), lambda i: (0, i))] * len(tabs),
        out_specs=[pl.BlockSpec((CB, D), lambda i: (i, 0))] * len(tabs),
        out_shape=[jax.ShapeDtypeStruct((grid * CB, D), dtype)] * len(tabs),
    )(*tabs)


@functools.lru_cache(maxsize=None)
def _build(B, NC, NS):
    NW = NC * NS
    BW = B // NW
    CH = 128
    NCHUNK = BW // CH
    GPC = CH // L

    mesh = plsc.VectorSubcoreMesh(core_axis_name="c", subcore_axis_name="s")

    @functools.partial(
        pl.kernel,
        out_type=jax.ShapeDtypeStruct((B,), jnp.float32),
        mesh=mesh,
        compiler_params=pltpu.CompilerParams(
            needs_layout_passes=False, use_tc_tiling_on_sc=False),
        scratch_types=[
            pltpu.VMEM((BW,), jnp.int32),
            pltpu.VMEM((BW,), jnp.int32),
            pltpu.VMEM((64, DIM), jnp.float32),
            pltpu.VMEM((DIM, DIM), jnp.float32),
            pltpu.VMEM((DIM,), jnp.float32),
            pltpu.VMEM((CH, NN), jnp.int32),
            pltpu.VMEM((CH, NN), jnp.int32),
            pltpu.VMEM((CH, NN), jnp.int32),
            pltpu.VMEM((CH, NN), jnp.int32),
            pltpu.VMEM((CH, DIM), jnp.float32),
            pltpu.VMEM((CH, DIM), jnp.float32),
            pltpu.VMEM((CH * NN,), jnp.int32),
            pltpu.VMEM((CH * NN,), jnp.int32),
            pltpu.VMEM((CH * NN,), jnp.int32),
            pltpu.VMEM((CH * NN, DIM), jnp.float32),
            pltpu.VMEM((CH * NN, DIM), jnp.float32),
            pltpu.VMEM((CH * NN, DIM), jnp.float32),
            pltpu.VMEM((BW,), jnp.float32),
            pltpu.SemaphoreType.DMA,
            pltpu.SemaphoreType.DMA,
            pltpu.SemaphoreType.DMA,
            pltpu.SemaphoreType.DMA,
            pltpu.SemaphoreType.DMA,
            pltpu.SemaphoreType.DMA,
        ],
    )
    def klgcn(usr_h, ent_h, rel_h, w_h, b_h, u2i_h, i2u_h, ae_h, ar_h, u_h,
              v_h, out_h,
              u_v, v_v, rel_v, w_v, b_v, nb_u2i, nb_i2u, nb_ae, nb_ar,
              ue_r, io_r, fl_i2u, fl_u2i, fl_ae,
              usr_nb, ent_nb1, ent_nb2, out_v,
              s0, s1, s2, s3, s4, s5):
        wid = lax.axis_index("s") * NC + lax.axis_index("c")
        base = wid * BW
        pltpu.sync_copy(u_h.at[pl.ds(base, BW)], u_v)
        pltpu.sync_copy(v_h.at[pl.ds(base, BW)], v_v)
        pltpu.sync_copy(rel_h, rel_v)
        pltpu.sync_copy(w_h, w_v)
        pltpu.sync_copy(b_h, b_v)
        iota = lax.iota(jnp.int32, L)

        def chunk_body(c, carry):
            off = c * CH
            uc = u_v.at[pl.ds(off, CH)]
            vc = v_v.at[pl.ds(off, CH)]
            d_u2i = pltpu.async_copy(u2i_h.at[uc], nb_u2i, s0)
            d_i2u = pltpu.async_copy(i2u_h.at[vc], nb_i2u, s1)
            d_ae = pltpu.async_copy(ae_h.at[vc], nb_ae, s2)
            d_ar = pltpu.async_copy(ar_h.at[vc], nb_ar, s3)
            d_ue = pltpu.async_copy(usr_h.at[uc], ue_r, s4)
            d_io = pltpu.async_copy(ent_h.at[vc], io_r, s5)
            d_u2i.wait()
            d_i2u.wait()
            d_ae.wait()
            riota = jnp.right_shift(iota, 3)
            ciota = jnp.bitwise_and(iota, 7)

            def flat_body(i, cf):
                ridx = riota + i * 2
                o = i * L
                fl_i2u[pl.ds(o, L)] = plsc.load_gather(nb_i2u, [ridx, ciota])
                fl_u2i[pl.ds(o, L)] = plsc.load_gather(nb_u2i, [ridx, ciota])
                fl_ae[pl.ds(o, L)] = plsc.load_gather(nb_ae, [ridx, ciota])
                return cf

            lax.fori_loop(0, CH * NN // L, flat_body, 0)
            e_usr = pltpu.async_copy(usr_h.at[fl_i2u], usr_nb, s1)
            e_ent1 = pltpu.async_copy(ent_h.at[fl_u2i], ent_nb1, s0)
            e_ent2 = pltpu.async_copy(ent_h.at[fl_ae], ent_nb2, s2)
            d_ar.wait()
            d_ue.wait()
            d_io.wait()
            e_usr.wait()
            e_ent1.wait()
            e_ent2.wait()

            def group_body(g, carry2):
                rows = g * L + iota
                frows = [rows * NN + _splat(j) for j in range(NN)]
                ue = [plsc.load_gather(ue_r, [rows, _splat(d)])
                      for d in range(DIM)]
                s_list = []
                for j in range(NN):
                    relid = plsc.load_gather(nb_ar, [rows, _splat(j)])
                    acc = ue[0] * plsc.load_gather(rel_v, [relid, _splat(0)])
                    for d in range(1, DIM):
                        acc = acc + ue[d] * plsc.load_gather(
                            rel_v, [relid, _splat(d)])
                    s_list.append(acc)
                m = s_list[0]
                for j in range(1, NN):
                    m = jnp.maximum(m, s_list[j])
                e_list = [jnp.exp(sj - m) for sj in s_list]
                tot = e_list[0]
                for j in range(1, NN):
                    tot = tot + e_list[j]
                inv = 1.0 / tot
                p_list = [ej * inv for ej in e_list]
                y = [None] * DIM
                for d in range(DIM):
                    a = p_list[0] * plsc.load_gather(
                        ent_nb2, [frows[0], _splat(d)])
                    for j in range(1, NN):
                        a = a + p_list[j] * plsc.load_gather(
                            ent_nb2, [frows[j], _splat(d)])
                    x_d = a + plsc.load_gather(io_r, [rows, _splat(d)])
                    for dp in range(DIM):
                        w_sc = plsc.load_gather(w_v, [_splat(d), _splat(dp)])
                        term = x_d * w_sc
                        y[dp] = term if y[dp] is None else y[dp] + term
                t_list = []
                for dp in range(DIM):
                    yv = y[dp] + plsc.load_gather(b_v, [_splat(dp)])
                    tt = jnp.exp(jnp.abs(yv) * (-2.0))
                    r = (1.0 - tt) / (1.0 + tt)
                    t_list.append(jnp.where(yv < 0.0, -r, r))
                sc = None
                for d in range(DIM):
                    lu = plsc.load_gather(usr_nb, [frows[0], _splat(d)])
                    for j in range(1, NN):
                        lu = lu + plsc.load_gather(
                            usr_nb, [frows[j], _splat(d)])
                    li = plsc.load_gather(ent_nb1, [frows[0], _splat(d)])
                    for j in range(1, NN):
                        li = li + plsc.load_gather(
                            ent_nb1, [frows[j], _splat(d)])
                    uf = 0.0625 * lu + 0.5 * ue[d]
                    if_ = 0.0625 * li + 0.5 * t_list[d]
                    term = uf * if_
                    sc = term if sc is None else sc + term
                sig = 1.0 / (1.0 + jnp.exp(-sc))
                out_v[pl.ds(off + g * L, L)] = sig
                return carry2

            lax.fori_loop(0, GPC, group_body, 0)
            return carry

        lax.fori_loop(0, NCHUNK, chunk_body, 0)
        pltpu.sync_copy(out_v, out_h.at[pl.ds(base, BW)])

    return klgcn


def kernel(usr, ent, rel, W, b, u2i, i2u, adj_ent, adj_rel, u, v):
    B = u.shape[0]
    usr_rm, ent_rm = _tc_transpose_many([usr.T, ent.T], DIM, jnp.float32)
    u2i_rm, i2u_rm, ae_rm, ar_rm = _tc_transpose_many(
        [u2i.T, i2u.T, adj_ent.T, adj_rel.T], NN, jnp.int32)
    info = plsc.get_sparse_core_info()
    fn = _build(B, info.num_cores, info.num_subcores)
    return fn(usr_rm, ent_rm, rel, W, b, u2i_rm, i2u_rm, ae_rm, ar_rm,
              u.astype(jnp.int32), v.astype(jnp.int32))


# speedup vs baseline: 7.0932x; 6.3657x over previous
import functools

import jax
import jax.numpy as jnp
from jax import lax
from jax.experimental import pallas as pl
from jax.experimental.pallas import tpu as pltpu
from jax.experimental.pallas import tpu_sc as plsc

DIM = 16
NN = 8
L = 16


def _splat(val):
    return jnp.full((L,), val, jnp.int32)


TCB = 8192
L2CB = 13


def _tc_relayout_many(tabs, D, dtype):
    n = tabs[0].shape[1]
    grid = (n + TCB - 1) // TCB
    g = 128 // D
    sub = TCB // g

    def body(*refs):
        k = len(refs) // 2
        for i_ref, o_ref in zip(refs[:k], refs[k:]):
            x = i_ref[...]
            x2 = jnp.concatenate(
                [lax.slice(x, (0, b * sub), (D, (b + 1) * sub))
                 for b in range(g)], axis=0)
            o_ref[...] = jnp.transpose(x2, (1, 0))

    outs = pl.pallas_call(
        body,
        grid=(grid,),
        in_specs=[pl.BlockSpec((D, TCB), lambda i: (0, i))] * len(tabs),
        out_specs=[pl.BlockSpec((sub, 128), lambda i: (i, 0))] * len(tabs),
        out_shape=[jax.ShapeDtypeStruct((grid * sub, 128), dtype)] * len(tabs),
    )(*tabs)
    return [o.reshape(grid * TCB, D) for o in outs]


def _perm16(ids):
    return (jnp.bitwise_and(ids, -TCB)
            + jnp.left_shift(jnp.bitwise_and(ids, 1023), 3)
            + jnp.right_shift(jnp.bitwise_and(ids, TCB - 1), 10))


def _perm8(ids):
    return (jnp.bitwise_and(ids, -TCB)
            + jnp.left_shift(jnp.bitwise_and(ids, 511), 4)
            + jnp.right_shift(jnp.bitwise_and(ids, TCB - 1), 9))


@functools.lru_cache(maxsize=None)
def _build(B, NC, NS):
    NW = NC * NS
    BW = B // NW
    CH = 128
    NCHUNK = BW // CH
    GPC = CH // L

    mesh = plsc.VectorSubcoreMesh(core_axis_name="c", subcore_axis_name="s")

    @functools.partial(
        pl.kernel,
        out_type=jax.ShapeDtypeStruct((B,), jnp.float32),
        mesh=mesh,
        compiler_params=pltpu.CompilerParams(
            needs_layout_passes=False, use_tc_tiling_on_sc=False),
        scratch_types=[
            pltpu.VMEM((BW,), jnp.int32),
            pltpu.VMEM((BW,), jnp.int32),
            pltpu.VMEM((BW,), jnp.int32),
            pltpu.VMEM((BW,), jnp.int32),
            pltpu.VMEM((BW,), jnp.int32),
            pltpu.VMEM((BW,), jnp.int32),
            pltpu.VMEM((64, DIM), jnp.float32),
            pltpu.VMEM((DIM, DIM), jnp.float32),
            pltpu.VMEM((DIM,), jnp.float32),
            pltpu.VMEM((CH, NN), jnp.int32),
            pltpu.VMEM((CH, NN), jnp.int32),
            pltpu.VMEM((CH, NN), jnp.int32),
            pltpu.VMEM((CH, NN), jnp.int32),
            pltpu.VMEM((CH, DIM), jnp.float32),
            pltpu.VMEM((CH, DIM), jnp.float32),
            pltpu.VMEM((CH * NN,), jnp.int32),
            pltpu.VMEM((CH * NN,), jnp.int32),
            pltpu.VMEM((CH * NN,), jnp.int32),
            pltpu.VMEM((CH * NN, DIM), jnp.float32),
            pltpu.VMEM((CH * NN, DIM), jnp.float32),
            pltpu.VMEM((CH * NN, DIM), jnp.float32),
            pltpu.VMEM((BW,), jnp.float32),
            pltpu.SemaphoreType.DMA,
            pltpu.SemaphoreType.DMA,
            pltpu.SemaphoreType.DMA,
            pltpu.SemaphoreType.DMA,
            pltpu.SemaphoreType.DMA,
            pltpu.SemaphoreType.DMA,
        ],
    )
    def klgcn(usr_h, ent_h, rel_h, w_h, b_h, u2i_h, i2u_h, ae_h, ar_h, u_h,
              v_h, out_h,
              u_v, v_v, pu8, pu16, pv8, pv16,
              rel_v, w_v, b_v, nb_u2i, nb_i2u, nb_ae, nb_ar,
              ue_r, io_r, fl_i2u, fl_u2i, fl_ae,
              usr_nb, ent_nb1, ent_nb2, out_v,
              s0, s1, s2, s3, s4, s5):
        wid = lax.axis_index("s") * NC + lax.axis_index("c")
        base = wid * BW
        pltpu.sync_copy(u_h.at[pl.ds(base, BW)], u_v)
        pltpu.sync_copy(v_h.at[pl.ds(base, BW)], v_v)
        pltpu.sync_copy(rel_h, rel_v)
        pltpu.sync_copy(w_h, w_v)
        pltpu.sync_copy(b_h, b_v)
        iota = lax.iota(jnp.int32, L)

        def perm_body(i, cp):
            o = i * L
            us = u_v[pl.ds(o, L)]
            vs = v_v[pl.ds(o, L)]
            pu8[pl.ds(o, L)] = _perm8(us)
            pu16[pl.ds(o, L)] = _perm16(us)
            pv8[pl.ds(o, L)] = _perm8(vs)
            pv16[pl.ds(o, L)] = _perm16(vs)
            return cp

        lax.fori_loop(0, BW // L, perm_body, 0)

        def chunk_body(c, carry):
            off = c * CH
            uc8 = pu8.at[pl.ds(off, CH)]
            uc16 = pu16.at[pl.ds(off, CH)]
            vc8 = pv8.at[pl.ds(off, CH)]
            vc16 = pv16.at[pl.ds(off, CH)]
            d_u2i = pltpu.async_copy(u2i_h.at[uc8], nb_u2i, s0)
            d_i2u = pltpu.async_copy(i2u_h.at[vc8], nb_i2u, s1)
            d_ae = pltpu.async_copy(ae_h.at[vc8], nb_ae, s2)
            d_ar = pltpu.async_copy(ar_h.at[vc8], nb_ar, s3)
            d_ue = pltpu.async_copy(usr_h.at[uc16], ue_r, s4)
            d_io = pltpu.async_copy(ent_h.at[vc16], io_r, s5)
            d_u2i.wait()
            d_i2u.wait()
            d_ae.wait()
            riota = jnp.right_shift(iota, 3)
            ciota = jnp.bitwise_and(iota, 7)

            def flat_body(i, cf):
                ridx = riota + i * 2
                o = i * L
                fl_i2u[pl.ds(o, L)] = _perm16(
                    plsc.load_gather(nb_i2u, [ridx, ciota]))
                fl_u2i[pl.ds(o, L)] = _perm16(
                    plsc.load_gather(nb_u2i, [ridx, ciota]))
                fl_ae[pl.ds(o, L)] = _perm16(
                    plsc.load_gather(nb_ae, [ridx, ciota]))
                return cf

            lax.fori_loop(0, CH * NN // L, flat_body, 0)
            e_usr = pltpu.async_copy(usr_h.at[fl_i2u], usr_nb, s1)
            e_ent1 = pltpu.async_copy(ent_h.at[fl_u2i], ent_nb1, s0)
            e_ent2 = pltpu.async_copy(ent_h.at[fl_ae], ent_nb2, s2)
            d_ar.wait()
            d_ue.wait()
            d_io.wait()
            e_usr.wait()
            e_ent1.wait()
            e_ent2.wait()

            def group_body(g, carry2):
                rows = g * L + iota
                frows = [rows * NN + _splat(j) for j in range(NN)]
                ue = [plsc.load_gather(ue_r, [rows, _splat(d)])
                      for d in range(DIM)]
                s_list = []
                for j in range(NN):
                    relid = plsc.load_gather(nb_ar, [rows, _splat(j)])
                    acc = ue[0] * plsc.load_gather(rel_v, [relid, _splat(0)])
                    for d in range(1, DIM):
                        acc = acc + ue[d] * plsc.load_gather(
                            rel_v, [relid, _splat(d)])
                    s_list.append(acc)
                m = s_list[0]
                for j in range(1, NN):
                    m = jnp.maximum(m, s_list[j])
                e_list = [jnp.exp(sj - m) for sj in s_list]
                tot = e_list[0]
                for j in range(1, NN):
                    tot = tot + e_list[j]
                inv = 1.0 / tot
                p_list = [ej * inv for ej in e_list]
                y = [None] * DIM
                for d in range(DIM):
                    a = p_list[0] * plsc.load_gather(
                        ent_nb2, [frows[0], _splat(d)])
                    for j in range(1, NN):
                        a = a + p_list[j] * plsc.load_gather(
                            ent_nb2, [frows[j], _splat(d)])
                    x_d = a + plsc.load_gather(io_r, [rows, _splat(d)])
                    for dp in range(DIM):
                        w_sc = plsc.load_gather(w_v, [_splat(d), _splat(dp)])
                        term = x_d * w_sc
                        y[dp] = term if y[dp] is None else y[dp] + term
                t_list = []
                for dp in range(DIM):
                    yv = y[dp] + plsc.load_gather(b_v, [_splat(dp)])
                    tt = jnp.exp(jnp.abs(yv) * (-2.0))
                    r = (1.0 - tt) / (1.0 + tt)
                    t_list.append(jnp.where(yv < 0.0, -r, r))
                sc = None
                for d in range(DIM):
                    lu = plsc.load_gather(usr_nb, [frows[0], _splat(d)])
                    for j in range(1, NN):
                        lu = lu + plsc.load_gather(
                            usr_nb, [frows[j], _splat(d)])
                    li = plsc.load_gather(ent_nb1, [frows[0], _splat(d)])
                    for j in range(1, NN):
                        li = li + plsc.load_gather(
                            ent_nb1, [frows[j], _splat(d)])
                    uf = 0.0625 * lu + 0.5 * ue[d]
                    if_ = 0.0625 * li + 0.5 * t_list[d]
                    term = uf * if_
                    sc = term if sc is None else sc + term
                sig = 1.0 / (1.0 + jnp.exp(-sc))
                out_v[pl.ds(off + g * L, L)] = sig
                return carry2

            lax.fori_loop(0, GPC, group_body, 0)
            return carry

        lax.fori_loop(0, NCHUNK, chunk_body, 0)
        pltpu.sync_copy(out_v, out_h.at[pl.ds(base, BW)])

    return klgcn


def kernel(usr, ent, rel, W, b, u2i, i2u, adj_ent, adj_rel, u, v):
    B = u.shape[0]
    usr_rm, ent_rm = _tc_relayout_many([usr.T, ent.T], DIM, jnp.float32)
    u2i_rm, i2u_rm, ae_rm, ar_rm = _tc_relayout_many(
        [u2i.T, i2u.T, adj_ent.T, adj_rel.T], NN, jnp.int32)
    info = plsc.get_sparse_core_info()
    fn = _build(B, info.num_cores, info.num_subcores)
    return fn(usr_rm, ent_rm, rel, W, b, u2i_rm, i2u_rm, ae_rm, ar_rm,
              u.astype(jnp.int32), v.astype(jnp.int32))


# speedup vs baseline: 9.1651x; 1.2921x over previous
import functools

import jax
import jax.numpy as jnp
from jax import lax
from jax.experimental import pallas as pl
from jax.experimental.pallas import tpu as pltpu
from jax.experimental.pallas import tpu_sc as plsc

DIM = 16
NN = 8
L = 16


def _splat(val):
    return jnp.full((L,), val, jnp.int32)


TCB = 8192
L2CB = 13


def _tree_sum(vals):
    while len(vals) > 1:
        vals = [a + b for a, b in zip(vals[::2], vals[1::2])]
    return vals[0]


def _tc_relayout_all(specs):
    n = specs[0][0].shape[1]
    grid = (n + TCB - 1) // TCB

    def body(*refs):
        k = len(refs) // 2
        for (tab, D, dt), i_ref, o_ref in zip(specs, refs[:k], refs[k:]):
            g = 128 // D
            sub = TCB // g
            x = i_ref[...]
            x2 = jnp.concatenate(
                [lax.slice(x, (0, b * sub), (D, (b + 1) * sub))
                 for b in range(g)], axis=0)
            o_ref[...] = jnp.transpose(x2, (1, 0))

    outs = pl.pallas_call(
        body,
        grid=(grid,),
        in_specs=[pl.BlockSpec((D, TCB), lambda i: (0, i))
                  for (_, D, _) in specs],
        out_specs=[pl.BlockSpec((TCB // (128 // D), 128), lambda i: (i, 0))
                   for (_, D, _) in specs],
        out_shape=[jax.ShapeDtypeStruct((grid * TCB // (128 // D), 128), dt)
                   for (_, D, dt) in specs],
    )(*[t for (t, _, _) in specs])
    return [o.reshape(grid * TCB, D) for o, (_, D, _) in zip(outs, specs)]


def _tc_relayout_many(tabs, D, dtype):
    n = tabs[0].shape[1]
    grid = (n + TCB - 1) // TCB
    g = 128 // D
    sub = TCB // g

    def body(*refs):
        k = len(refs) // 2
        for i_ref, o_ref in zip(refs[:k], refs[k:]):
            x = i_ref[...]
            x2 = jnp.concatenate(
                [lax.slice(x, (0, b * sub), (D, (b + 1) * sub))
                 for b in range(g)], axis=0)
            o_ref[...] = jnp.transpose(x2, (1, 0))

    outs = pl.pallas_call(
        body,
        grid=(grid,),
        in_specs=[pl.BlockSpec((D, TCB), lambda i: (0, i))] * len(tabs),
        out_specs=[pl.BlockSpec((sub, 128), lambda i: (i, 0))] * len(tabs),
        out_shape=[jax.ShapeDtypeStruct((grid * sub, 128), dtype)] * len(tabs),
    )(*tabs)
    return [o.reshape(grid * TCB, D) for o in outs]


def _perm16(ids):
    return (jnp.bitwise_and(ids, -TCB)
            + jnp.left_shift(jnp.bitwise_and(ids, 1023), 3)
            + jnp.right_shift(jnp.bitwise_and(ids, TCB - 1), 10))


def _perm8(ids):
    return (jnp.bitwise_and(ids, -TCB)
            + jnp.left_shift(jnp.bitwise_and(ids, 511), 4)
            + jnp.right_shift(jnp.bitwise_and(ids, TCB - 1), 9))


@functools.lru_cache(maxsize=None)
def _build(B, NC, NS):
    NW = NC * NS
    BW = B // NW
    CH = 128
    NCHUNK = BW // CH
    GPC = CH // L

    mesh = plsc.VectorSubcoreMesh(core_axis_name="c", subcore_axis_name="s")

    @functools.partial(
        pl.kernel,
        out_type=jax.ShapeDtypeStruct((B,), jnp.float32),
        mesh=mesh,
        compiler_params=pltpu.CompilerParams(
            needs_layout_passes=False, use_tc_tiling_on_sc=False),
        scratch_types=[
            pltpu.VMEM((BW,), jnp.int32),
            pltpu.VMEM((BW,), jnp.int32),
            pltpu.VMEM((BW,), jnp.int32),
            pltpu.VMEM((BW,), jnp.int32),
            pltpu.VMEM((BW,), jnp.int32),
            pltpu.VMEM((BW,), jnp.int32),
            pltpu.VMEM((64, DIM), jnp.float32),
            pltpu.VMEM((DIM, DIM), jnp.float32),
            pltpu.VMEM((DIM,), jnp.float32),
            pltpu.VMEM((CH, NN), jnp.int32),
            pltpu.VMEM((CH, NN), jnp.int32),
            pltpu.VMEM((CH, NN), jnp.int32),
            pltpu.VMEM((CH, NN), jnp.int32),
            pltpu.VMEM((CH, DIM), jnp.float32),
            pltpu.VMEM((CH, DIM), jnp.float32),
            pltpu.VMEM((CH * NN,), jnp.int32),
            pltpu.VMEM((CH * NN,), jnp.int32),
            pltpu.VMEM((CH * NN,), jnp.int32),
            pltpu.VMEM((CH * NN, DIM), jnp.float32),
            pltpu.VMEM((CH * NN, DIM), jnp.float32),
            pltpu.VMEM((CH * NN, DIM), jnp.float32),
            pltpu.VMEM((BW,), jnp.float32),
            pltpu.SemaphoreType.DMA,
            pltpu.SemaphoreType.DMA,
            pltpu.SemaphoreType.DMA,
            pltpu.SemaphoreType.DMA,
            pltpu.SemaphoreType.DMA,
            pltpu.SemaphoreType.DMA,
        ],
    )
    def klgcn(usr_h, ent_h, rel_h, w_h, b_h, u2i_h, i2u_h, ae_h, ar_h, u_h,
              v_h, out_h,
              u_v, v_v, pu8, pu16, pv8, pv16,
              rel_v, w_v, b_v, nb_u2i, nb_i2u, nb_ae, nb_ar,
              ue_r, io_r, fl_i2u, fl_u2i, fl_ae,
              usr_nb, ent_nb1, ent_nb2, out_v,
              s0, s1, s2, s3, s4, s5):
        wid = lax.axis_index("s") * NC + lax.axis_index("c")
        base = wid * BW
        pltpu.sync_copy(u_h.at[pl.ds(base, BW)], u_v)
        pltpu.sync_copy(v_h.at[pl.ds(base, BW)], v_v)
        pltpu.sync_copy(rel_h, rel_v)
        pltpu.sync_copy(w_h, w_v)
        pltpu.sync_copy(b_h, b_v)
        iota = lax.iota(jnp.int32, L)

        def perm_body(i, cp):
            o = i * L
            us = u_v[pl.ds(o, L)]
            vs = v_v[pl.ds(o, L)]
            pu8[pl.ds(o, L)] = _perm8(us)
            pu16[pl.ds(o, L)] = _perm16(us)
            pv8[pl.ds(o, L)] = _perm8(vs)
            pv16[pl.ds(o, L)] = _perm16(vs)
            return cp

        lax.fori_loop(0, BW // L, perm_body, 0)

        def chunk_body(c, carry):
            off = c * CH
            uc8 = pu8.at[pl.ds(off, CH)]
            uc16 = pu16.at[pl.ds(off, CH)]
            vc8 = pv8.at[pl.ds(off, CH)]
            vc16 = pv16.at[pl.ds(off, CH)]
            d_u2i = pltpu.async_copy(u2i_h.at[uc8], nb_u2i, s0)
            d_i2u = pltpu.async_copy(i2u_h.at[vc8], nb_i2u, s1)
            d_ae = pltpu.async_copy(ae_h.at[vc8], nb_ae, s2)
            d_ar = pltpu.async_copy(ar_h.at[vc8], nb_ar, s3)
            d_ue = pltpu.async_copy(usr_h.at[uc16], ue_r, s4)
            d_io = pltpu.async_copy(ent_h.at[vc16], io_r, s5)
            d_u2i.wait()
            d_i2u.wait()
            d_ae.wait()
            riota = jnp.right_shift(iota, 3)
            ciota = jnp.bitwise_and(iota, 7)

            def flat_body(i, cf):
                ridx = riota + i * 2
                o = i * L
                fl_i2u[pl.ds(o, L)] = _perm16(
                    plsc.load_gather(nb_i2u, [ridx, ciota]))
                fl_u2i[pl.ds(o, L)] = _perm16(
                    plsc.load_gather(nb_u2i, [ridx, ciota]))
                fl_ae[pl.ds(o, L)] = _perm16(
                    plsc.load_gather(nb_ae, [ridx, ciota]))
                return cf

            lax.fori_loop(0, CH * NN // L, flat_body, 0)
            e_usr = pltpu.async_copy(usr_h.at[fl_i2u], usr_nb, s1)
            e_ent1 = pltpu.async_copy(ent_h.at[fl_u2i], ent_nb1, s0)
            e_ent2 = pltpu.async_copy(ent_h.at[fl_ae], ent_nb2, s2)
            d_ar.wait()
            d_ue.wait()
            d_io.wait()
            e_usr.wait()
            e_ent1.wait()
            e_ent2.wait()

            def group_body(g, carry2):
                rows = g * L + iota
                frows = [rows * NN + _splat(j) for j in range(NN)]
                relids = [plsc.load_gather(nb_ar, [rows, _splat(j)])
                          for j in range(NN)]
                s_list = [None] * NN
                for d in range(DIM):
                    ue_d = plsc.load_gather(ue_r, [rows, _splat(d)])
                    for j in range(NN):
                        term = ue_d * plsc.load_gather(
                            rel_v, [relids[j], _splat(d)])
                        s_list[j] = term if s_list[j] is None \
                            else s_list[j] + term
                m = s_list[0]
                for j in range(1, NN):
                    m = jnp.maximum(m, s_list[j])
                e_list = [jnp.exp(sj - m) for sj in s_list]
                tot = _tree_sum(list(e_list))
                inv = 1.0 / tot
                p_list = [ej * inv for ej in e_list]
                y = [None] * DIM
                for d in range(DIM):
                    gs = [p_list[j] * plsc.load_gather(
                        ent_nb2, [frows[j], _splat(d)]) for j in range(NN)]
                    x_d = _tree_sum(gs) + plsc.load_gather(
                        io_r, [rows, _splat(d)])
                    for dp in range(DIM):
                        w_sc = plsc.load_gather(w_v, [_splat(d), _splat(dp)])
                        term = x_d * w_sc
                        y[dp] = term if y[dp] is None else y[dp] + term
                sc = None
                for dp in range(DIM):
                    yv = y[dp] + plsc.load_gather(b_v, [_splat(dp)])
                    tt = jnp.exp(jnp.abs(yv) * (-2.0))
                    r = (1.0 - tt) / (1.0 + tt)
                    t_d = jnp.where(yv < 0.0, -r, r)
                    lu = _tree_sum([plsc.load_gather(usr_nb, [frows[j],
                                                             _splat(dp)])
                                    for j in range(NN)])
                    li = _tree_sum([plsc.load_gather(ent_nb1, [frows[j],
                                                              _splat(dp)])
                                    for j in range(NN)])
                    ue_d = plsc.load_gather(ue_r, [rows, _splat(dp)])
                    uf = 0.0625 * lu + 0.5 * ue_d
                    if_ = 0.0625 * li + 0.5 * t_d
                    term = uf * if_
                    sc = term if sc is None else sc + term
                sig = 1.0 / (1.0 + jnp.exp(-sc))
                out_v[pl.ds(off + g * L, L)] = sig
                return carry2

            lax.fori_loop(0, GPC, group_body, 0)
            return carry

        lax.fori_loop(0, NCHUNK, chunk_body, 0)
        pltpu.sync_copy(out_v, out_h.at[pl.ds(base, BW)])

    return klgcn


def kernel(usr, ent, rel, W, b, u2i, i2u, adj_ent, adj_rel, u, v):
    B = u.shape[0]
    usr_rm, ent_rm, u2i_rm, i2u_rm, ae_rm, ar_rm = _tc_relayout_all([
        (usr.T, DIM, jnp.float32), (ent.T, DIM, jnp.float32),
        (u2i.T, NN, jnp.int32), (i2u.T, NN, jnp.int32),
        (adj_ent.T, NN, jnp.int32), (adj_rel.T, NN, jnp.int32)])
    info = plsc.get_sparse_core_info()
    fn = _build(B, info.num_cores, info.num_subcores)
    return fn(usr_rm, ent_rm, rel, W, b, u2i_rm, i2u_rm, ae_rm, ar_rm,
              u.astype(jnp.int32), v.astype(jnp.int32))


# speedup vs baseline: 9.5184x; 1.0385x over previous
import functools

import jax
import jax.numpy as jnp
from jax import lax
from jax.experimental import pallas as pl
from jax.experimental.pallas import tpu as pltpu
from jax.experimental.pallas import tpu_sc as plsc

DIM = 16
NN = 8
L = 16


def _splat(val):
    return jnp.full((L,), val, jnp.int32)


TCB = 8192
L2CB = 13


def _tree_sum(vals):
    while len(vals) > 1:
        vals = [a + b for a, b in zip(vals[::2], vals[1::2])]
    return vals[0]


def _tc_relayout_all(usr_t, ent_t, u2i_t, i2u_t, ae_t, ar_t):
    n = usr_t.shape[1]
    grid = (n + TCB - 1) // TCB
    out_ds = [DIM, DIM, NN, NN, NN]

    def relay(x, D):
        g = 128 // D
        sub = TCB // g
        return jnp.transpose(jnp.concatenate(
            [lax.slice(x, (0, b * sub), (D, (b + 1) * sub))
             for b in range(g)], axis=0), (1, 0))

    def body(u_i, e_i, a_i, b_i, ae_i, ar_i, u_o, e_o, a_o, b_o, p_o):
        u_o[...] = relay(u_i[...], DIM)
        e_o[...] = relay(e_i[...], DIM)
        a_o[...] = relay(a_i[...], NN)
        b_o[...] = relay(b_i[...], NN)
        packed = jnp.bitwise_or(ae_i[...], jnp.left_shift(ar_i[...], 20))
        p_o[...] = relay(packed, NN)

    in_ds = [DIM, DIM, NN, NN, NN, NN]
    outs = pl.pallas_call(
        body,
        grid=(grid,),
        in_specs=[pl.BlockSpec((D, TCB), lambda i: (0, i)) for D in in_ds],
        out_specs=[pl.BlockSpec((TCB // (128 // D), 128), lambda i: (i, 0))
                   for D in out_ds],
        out_shape=[jax.ShapeDtypeStruct(
            (grid * TCB // (128 // D), 128),
            jnp.float32 if D == DIM else jnp.int32) for D in out_ds],
    )(usr_t, ent_t, u2i_t, i2u_t, ae_t, ar_t)
    return [o.reshape(grid * TCB, D) for o, D in zip(outs, out_ds)]


def _tc_relayout_many(tabs, D, dtype):
    n = tabs[0].shape[1]
    grid = (n + TCB - 1) // TCB
    g = 128 // D
    sub = TCB // g

    def body(*refs):
        k = len(refs) // 2
        for i_ref, o_ref in zip(refs[:k], refs[k:]):
            x = i_ref[...]
            x2 = jnp.concatenate(
                [lax.slice(x, (0, b * sub), (D, (b + 1) * sub))
                 for b in range(g)], axis=0)
            o_ref[...] = jnp.transpose(x2, (1, 0))

    outs = pl.pallas_call(
        body,
        grid=(grid,),
        in_specs=[pl.BlockSpec((D, TCB), lambda i: (0, i))] * len(tabs),
        out_specs=[pl.BlockSpec((sub, 128), lambda i: (i, 0))] * len(tabs),
        out_shape=[jax.ShapeDtypeStruct((grid * sub, 128), dtype)] * len(tabs),
    )(*tabs)
    return [o.reshape(grid * TCB, D) for o in outs]


def _perm16(ids):
    return (jnp.bitwise_and(ids, -TCB)
            + jnp.left_shift(jnp.bitwise_and(ids, 1023), 3)
            + jnp.right_shift(jnp.bitwise_and(ids, TCB - 1), 10))


def _perm8(ids):
    return (jnp.bitwise_and(ids, -TCB)
            + jnp.left_shift(jnp.bitwise_and(ids, 511), 4)
            + jnp.right_shift(jnp.bitwise_and(ids, TCB - 1), 9))


@functools.lru_cache(maxsize=None)
def _build(B, NC, NS):
    NW = NC * NS
    BW = B // NW
    CH = 128
    NCHUNK = BW // CH
    GPC = CH // L

    mesh = plsc.VectorSubcoreMesh(core_axis_name="c", subcore_axis_name="s")

    @functools.partial(
        pl.kernel,
        out_type=jax.ShapeDtypeStruct((B,), jnp.float32),
        mesh=mesh,
        compiler_params=pltpu.CompilerParams(
            needs_layout_passes=False, use_tc_tiling_on_sc=False),
        scratch_types=[
            pltpu.VMEM((BW,), jnp.int32),
            pltpu.VMEM((BW,), jnp.int32),
            pltpu.VMEM((BW,), jnp.int32),
            pltpu.VMEM((BW,), jnp.int32),
            pltpu.VMEM((BW,), jnp.int32),
            pltpu.VMEM((BW,), jnp.int32),
            pltpu.VMEM((64, DIM), jnp.float32),
            pltpu.VMEM((DIM, DIM), jnp.float32),
            pltpu.VMEM((DIM,), jnp.float32),
            pltpu.VMEM((CH, NN), jnp.int32),
            pltpu.VMEM((CH, NN), jnp.int32),
            pltpu.VMEM((CH, NN), jnp.int32),
            pltpu.VMEM((CH, DIM), jnp.float32),
            pltpu.VMEM((CH, DIM), jnp.float32),
            pltpu.VMEM((CH * NN,), jnp.int32),
            pltpu.VMEM((CH * NN,), jnp.int32),
            pltpu.VMEM((CH * NN,), jnp.int32),
            pltpu.VMEM((CH * NN,), jnp.int32),
            pltpu.VMEM((CH * NN, DIM), jnp.float32),
            pltpu.VMEM((CH * NN, DIM), jnp.float32),
            pltpu.VMEM((CH * NN, DIM), jnp.float32),
            pltpu.VMEM((BW,), jnp.float32),
            pltpu.SemaphoreType.DMA,
            pltpu.SemaphoreType.DMA,
            pltpu.SemaphoreType.DMA,
            pltpu.SemaphoreType.DMA,
            pltpu.SemaphoreType.DMA,
            pltpu.SemaphoreType.DMA,
        ],
    )
    def klgcn(usr_h, ent_h, rel_h, w_h, b_h, u2i_h, i2u_h, aepk_h, u_h,
              v_h, out_h,
              u_v, v_v, pu8, pu16, pv8, pv16,
              rel_v, w_v, b_v, nb_u2i, nb_i2u, nb_ae,
              ue_r, io_r, fl_i2u, fl_u2i, fl_ae, fl_ar,
              usr_nb, ent_nb1, ent_nb2, out_v,
              s0, s1, s2, s3, s4, s5):
        wid = lax.axis_index("s") * NC + lax.axis_index("c")
        base = wid * BW
        pltpu.sync_copy(u_h.at[pl.ds(base, BW)], u_v)
        pltpu.sync_copy(v_h.at[pl.ds(base, BW)], v_v)
        pltpu.sync_copy(rel_h, rel_v)
        pltpu.sync_copy(w_h, w_v)
        pltpu.sync_copy(b_h, b_v)
        iota = lax.iota(jnp.int32, L)

        def perm_body(i, cp):
            o = i * L
            us = u_v[pl.ds(o, L)]
            vs = v_v[pl.ds(o, L)]
            pu8[pl.ds(o, L)] = _perm8(us)
            pu16[pl.ds(o, L)] = _perm16(us)
            pv8[pl.ds(o, L)] = _perm8(vs)
            pv16[pl.ds(o, L)] = _perm16(vs)
            return cp

        lax.fori_loop(0, BW // L, perm_body, 0)

        def chunk_body(c, carry):
            off = c * CH
            uc8 = pu8.at[pl.ds(off, CH)]
            uc16 = pu16.at[pl.ds(off, CH)]
            vc8 = pv8.at[pl.ds(off, CH)]
            vc16 = pv16.at[pl.ds(off, CH)]
            d_u2i = pltpu.async_copy(u2i_h.at[uc8], nb_u2i, s0)
            d_i2u = pltpu.async_copy(i2u_h.at[vc8], nb_i2u, s1)
            d_ae = pltpu.async_copy(aepk_h.at[vc8], nb_ae, s2)
            d_ue = pltpu.async_copy(usr_h.at[uc16], ue_r, s4)
            d_io = pltpu.async_copy(ent_h.at[vc16], io_r, s5)
            d_u2i.wait()
            d_i2u.wait()
            d_ae.wait()
            riota = jnp.right_shift(iota, 3)
            ciota = jnp.bitwise_and(iota, 7)

            def flat_body(i, cf):
                ridx = riota + i * 2
                o = i * L
                fl_i2u[pl.ds(o, L)] = _perm16(
                    plsc.load_gather(nb_i2u, [ridx, ciota]))
                fl_u2i[pl.ds(o, L)] = _perm16(
                    plsc.load_gather(nb_u2i, [ridx, ciota]))
                pk = plsc.load_gather(nb_ae, [ridx, ciota])
                fl_ae[pl.ds(o, L)] = _perm16(
                    jnp.bitwise_and(pk, (1 << 20) - 1))
                fl_ar[pl.ds(o, L)] = jnp.right_shift(pk, 20)
                return cf

            lax.fori_loop(0, CH * NN // L, flat_body, 0)
            e_usr = pltpu.async_copy(usr_h.at[fl_i2u], usr_nb, s1)
            e_ent1 = pltpu.async_copy(ent_h.at[fl_u2i], ent_nb1, s0)
            e_ent2 = pltpu.async_copy(ent_h.at[fl_ae], ent_nb2, s2)
            d_ue.wait()
            d_io.wait()
            e_usr.wait()
            e_ent1.wait()
            e_ent2.wait()

            def group_body(g, carry2):
                rows = g * L + iota
                frows = [rows * NN + _splat(j) for j in range(NN)]
                relids = [plsc.load_gather(fl_ar, [frows[j]])
                          for j in range(NN)]
                s_list = [None] * NN
                for d in range(DIM):
                    ue_d = plsc.load_gather(ue_r, [rows, _splat(d)])
                    for j in range(NN):
                        term = ue_d * plsc.load_gather(
                            rel_v, [relids[j], _splat(d)])
                        s_list[j] = term if s_list[j] is None \
                            else s_list[j] + term
                m = s_list[0]
                for j in range(1, NN):
                    m = jnp.maximum(m, s_list[j])
                e_list = [jnp.exp(sj - m) for sj in s_list]
                tot = _tree_sum(list(e_list))
                inv = 1.0 / tot
                p_list = [ej * inv for ej in e_list]
                y = [None] * DIM
                for d in range(DIM):
                    gs = [p_list[j] * plsc.load_gather(
                        ent_nb2, [frows[j], _splat(d)]) for j in range(NN)]
                    x_d = _tree_sum(gs) + plsc.load_gather(
                        io_r, [rows, _splat(d)])
                    for dp in range(DIM):
                        w_sc = plsc.load_gather(w_v, [_splat(d), _splat(dp)])
                        term = x_d * w_sc
                        y[dp] = term if y[dp] is None else y[dp] + term
                sc = None
                for dp in range(DIM):
                    yv = y[dp] + plsc.load_gather(b_v, [_splat(dp)])
                    tt = jnp.exp(jnp.abs(yv) * (-2.0))
                    r = (1.0 - tt) / (1.0 + tt)
                    t_d = jnp.where(yv < 0.0, -r, r)
                    lu = _tree_sum([plsc.load_gather(usr_nb, [frows[j],
                                                             _splat(dp)])
                                    for j in range(NN)])
                    li = _tree_sum([plsc.load_gather(ent_nb1, [frows[j],
                                                              _splat(dp)])
                                    for j in range(NN)])
                    ue_d = plsc.load_gather(ue_r, [rows, _splat(dp)])
                    uf = 0.0625 * lu + 0.5 * ue_d
                    if_ = 0.0625 * li + 0.5 * t_d
                    term = uf * if_
                    sc = term if sc is None else sc + term
                sig = 1.0 / (1.0 + jnp.exp(-sc))
                out_v[pl.ds(off + g * L, L)] = sig
                return carry2

            lax.fori_loop(0, GPC, group_body, 0)
            return carry

        lax.fori_loop(0, NCHUNK, chunk_body, 0)
        pltpu.sync_copy(out_v, out_h.at[pl.ds(base, BW)])

    return klgcn


def kernel(usr, ent, rel, W, b, u2i, i2u, adj_ent, adj_rel, u, v):
    B = u.shape[0]
    usr_rm, ent_rm, u2i_rm, i2u_rm, aepk_rm = _tc_relayout_all(
        usr.T, ent.T, u2i.T, i2u.T, adj_ent.T, adj_rel.T)
    info = plsc.get_sparse_core_info()
    fn = _build(B, info.num_cores, info.num_subcores)
    return fn(usr_rm, ent_rm, rel, W, b, u2i_rm, i2u_rm, aepk_rm,
              u.astype(jnp.int32), v.astype(jnp.int32))


# speedup vs baseline: 9.6772x; 1.0167x over previous
import functools

import jax
import jax.numpy as jnp
from jax import lax
from jax.experimental import pallas as pl
from jax.experimental.pallas import tpu as pltpu
from jax.experimental.pallas import tpu_sc as plsc

DIM = 16
NN = 8
L = 16


def _splat(val):
    return jnp.full((L,), val, jnp.int32)


TCB = 8192
L2CB = 13


def _tree_sum(vals):
    while len(vals) > 1:
        vals = [a + b for a, b in zip(vals[::2], vals[1::2])]
    return vals[0]


def _tc_relayout_all(usr_t, ent_t, u2i_t, i2u_t, ae_t, ar_t):
    n = usr_t.shape[1]
    grid = (n + TCB - 1) // TCB
    out_ds = [DIM, DIM, NN, NN, NN]

    def relay(x, D):
        g = 128 // D
        sub = TCB // g
        return jnp.transpose(jnp.concatenate(
            [lax.slice(x, (0, b * sub), (D, (b + 1) * sub))
             for b in range(g)], axis=0), (1, 0))

    def body(u_i, e_i, a_i, b_i, ae_i, ar_i, u_o, e_o, a_o, b_o, p_o):
        u_o[...] = relay(u_i[...], DIM)
        e_o[...] = relay(e_i[...], DIM)
        a_o[...] = relay(a_i[...], NN)
        b_o[...] = relay(b_i[...], NN)
        packed = jnp.bitwise_or(ae_i[...], jnp.left_shift(ar_i[...], 20))
        p_o[...] = relay(packed, NN)

    in_ds = [DIM, DIM, NN, NN, NN, NN]
    outs = pl.pallas_call(
        body,
        grid=(grid,),
        in_specs=[pl.BlockSpec((D, TCB), lambda i: (0, i)) for D in in_ds],
        out_specs=[pl.BlockSpec((TCB // (128 // D), 128), lambda i: (i, 0))
                   for D in out_ds],
        out_shape=[jax.ShapeDtypeStruct(
            (grid * TCB // (128 // D), 128),
            jnp.float32 if D == DIM else jnp.int32) for D in out_ds],
    )(usr_t, ent_t, u2i_t, i2u_t, ae_t, ar_t)
    return [o.reshape(grid * TCB, D) for o, D in zip(outs, out_ds)]


def _tc_relayout_many(tabs, D, dtype):
    n = tabs[0].shape[1]
    grid = (n + TCB - 1) // TCB
    g = 128 // D
    sub = TCB // g

    def body(*refs):
        k = len(refs) // 2
        for i_ref, o_ref in zip(refs[:k], refs[k:]):
            x = i_ref[...]
            x2 = jnp.concatenate(
                [lax.slice(x, (0, b * sub), (D, (b + 1) * sub))
                 for b in range(g)], axis=0)
            o_ref[...] = jnp.transpose(x2, (1, 0))

    outs = pl.pallas_call(
        body,
        grid=(grid,),
        in_specs=[pl.BlockSpec((D, TCB), lambda i: (0, i))] * len(tabs),
        out_specs=[pl.BlockSpec((sub, 128), lambda i: (i, 0))] * len(tabs),
        out_shape=[jax.ShapeDtypeStruct((grid * sub, 128), dtype)] * len(tabs),
    )(*tabs)
    return [o.reshape(grid * TCB, D) for o in outs]


def _perm16(ids):
    return (jnp.bitwise_and(ids, -TCB)
            + jnp.left_shift(jnp.bitwise_and(ids, 1023), 3)
            + jnp.right_shift(jnp.bitwise_and(ids, TCB - 1), 10))


def _perm8(ids):
    return (jnp.bitwise_and(ids, -TCB)
            + jnp.left_shift(jnp.bitwise_and(ids, 511), 4)
            + jnp.right_shift(jnp.bitwise_and(ids, TCB - 1), 9))


@functools.lru_cache(maxsize=None)
def _build(B, NC, NS):
    NW = NC * NS
    BW = B // NW
    CH = 64
    NCHUNK = BW // CH
    GPC = CH // L

    mesh = plsc.VectorSubcoreMesh(core_axis_name="c", subcore_axis_name="s")

    @functools.partial(
        pl.kernel,
        out_type=jax.ShapeDtypeStruct((B,), jnp.float32),
        mesh=mesh,
        compiler_params=pltpu.CompilerParams(
            needs_layout_passes=False, use_tc_tiling_on_sc=False),
        scratch_types=[
            pltpu.VMEM((BW,), jnp.int32),
            pltpu.VMEM((BW,), jnp.int32),
            pltpu.VMEM((BW,), jnp.int32),
            pltpu.VMEM((BW,), jnp.int32),
            pltpu.VMEM((BW,), jnp.int32),
            pltpu.VMEM((BW,), jnp.int32),
            pltpu.VMEM((64, DIM), jnp.float32),
            pltpu.VMEM((DIM, DIM), jnp.float32),
            pltpu.VMEM((DIM,), jnp.float32),
            pltpu.VMEM((CH, NN), jnp.int32),
            pltpu.VMEM((CH, NN), jnp.int32),
            pltpu.VMEM((CH, NN), jnp.int32),
            pltpu.VMEM((2 * CH, DIM), jnp.float32),
            pltpu.VMEM((2 * CH, DIM), jnp.float32),
            pltpu.VMEM((2 * CH * NN,), jnp.int32),
            pltpu.VMEM((2 * CH * NN,), jnp.int32),
            pltpu.VMEM((2 * CH * NN,), jnp.int32),
            pltpu.VMEM((2 * CH * NN,), jnp.int32),
            pltpu.VMEM((2 * CH * NN, DIM), jnp.float32),
            pltpu.VMEM((2 * CH * NN, DIM), jnp.float32),
            pltpu.VMEM((2 * CH * NN, DIM), jnp.float32),
            pltpu.VMEM((BW,), jnp.float32),
        ] + [pltpu.SemaphoreType.DMA] * 14,
    )
    def klgcn(usr_h, ent_h, rel_h, w_h, b_h, u2i_h, i2u_h, aepk_h, u_h,
              v_h, out_h,
              u_v, v_v, pu8, pu16, pv8, pv16,
              rel_v, w_v, b_v, nb_u2i, nb_i2u, nb_ae,
              ue_r, io_r, fl_i2u, fl_u2i, fl_ae, fl_ar,
              usr_nb, ent_nb1, ent_nb2, out_v,
              s0, s1, s2, s3, s4, s5, s6, s7, s8, s9, s10, s11, s12, s13):
        wid = lax.axis_index("s") * NC + lax.axis_index("c")
        base = wid * BW
        pltpu.sync_copy(u_h.at[pl.ds(base, BW)], u_v)
        pltpu.sync_copy(v_h.at[pl.ds(base, BW)], v_v)
        pltpu.sync_copy(rel_h, rel_v)
        pltpu.sync_copy(w_h, w_v)
        pltpu.sync_copy(b_h, b_v)
        iota = lax.iota(jnp.int32, L)

        def perm_body(i, cp):
            o = i * L
            us = u_v[pl.ds(o, L)]
            vs = v_v[pl.ds(o, L)]
            pu8[pl.ds(o, L)] = _perm8(us)
            pu16[pl.ds(o, L)] = _perm16(us)
            pv8[pl.ds(o, L)] = _perm8(vs)
            pv16[pl.ds(o, L)] = _perm16(vs)
            return cp

        lax.fori_loop(0, BW // L, perm_body, 0)

        CHN = CH * NN
        riota = jnp.right_shift(iota, 3)
        ciota = jnp.bitwise_and(iota, 7)

        def issue_nb(c):
            off = c * CH
            pltpu.async_copy(u2i_h.at[pu8.at[pl.ds(off, CH)]], nb_u2i, s0)
            pltpu.async_copy(i2u_h.at[pv8.at[pl.ds(off, CH)]], nb_i2u, s1)
            pltpu.async_copy(aepk_h.at[pv8.at[pl.ds(off, CH)]], nb_ae, s2)

        def wait_nb():
            pltpu.make_async_copy(
                u2i_h.at[pu8.at[pl.ds(0, CH)]], nb_u2i, s0).wait()
            pltpu.make_async_copy(
                i2u_h.at[pv8.at[pl.ds(0, CH)]], nb_i2u, s1).wait()
            pltpu.make_async_copy(
                aepk_h.at[pv8.at[pl.ds(0, CH)]], nb_ae, s2).wait()

        ue_sems = ((s4, s5), (s6, s7))

        def issue_ueio(c, par):
            off = c * CH
            su, si = ue_sems[par]
            pltpu.async_copy(usr_h.at[pu16.at[pl.ds(off, CH)]],
                             ue_r.at[pl.ds(par * CH, CH)], su)
            pltpu.async_copy(ent_h.at[pv16.at[pl.ds(off, CH)]],
                             io_r.at[pl.ds(par * CH, CH)], si)

        def wait_ueio(par):
            su, si = ue_sems[par]
            pltpu.make_async_copy(usr_h.at[pu16.at[pl.ds(0, CH)]],
                                  ue_r.at[pl.ds(par * CH, CH)], su).wait()
            pltpu.make_async_copy(ent_h.at[pv16.at[pl.ds(0, CH)]],
                                  io_r.at[pl.ds(par * CH, CH)], si).wait()

        l2_sems = ((s8, s9, s10), (s11, s12, s13))

        def flatten(par):
            ob = par * CHN

            def flat_body(i, cf):
                ridx = riota + i * 2
                o = ob + i * L
                fl_i2u[pl.ds(o, L)] = _perm16(
                    plsc.load_gather(nb_i2u, [ridx, ciota]))
                fl_u2i[pl.ds(o, L)] = _perm16(
                    plsc.load_gather(nb_u2i, [ridx, ciota]))
                pk = plsc.load_gather(nb_ae, [ridx, ciota])
                fl_ae[pl.ds(o, L)] = _perm16(
                    jnp.bitwise_and(pk, (1 << 20) - 1))
                fl_ar[pl.ds(o, L)] = jnp.right_shift(pk, 20)
                return cf

            lax.fori_loop(0, CHN // L, flat_body, 0)

        def issue_l2(par):
            ob = par * CHN
            sa, sb, sc2 = l2_sems[par]
            pltpu.async_copy(usr_h.at[fl_i2u.at[pl.ds(ob, CHN)]],
                             usr_nb.at[pl.ds(ob, CHN)], sa)
            pltpu.async_copy(ent_h.at[fl_u2i.at[pl.ds(ob, CHN)]],
                             ent_nb1.at[pl.ds(ob, CHN)], sb)
            pltpu.async_copy(ent_h.at[fl_ae.at[pl.ds(ob, CHN)]],
                             ent_nb2.at[pl.ds(ob, CHN)], sc2)

        def wait_l2(par):
            ob = par * CHN
            sa, sb, sc2 = l2_sems[par]
            pltpu.make_async_copy(usr_h.at[fl_i2u.at[pl.ds(ob, CHN)]],
                                  usr_nb.at[pl.ds(ob, CHN)], sa).wait()
            pltpu.make_async_copy(ent_h.at[fl_u2i.at[pl.ds(ob, CHN)]],
                                  ent_nb1.at[pl.ds(ob, CHN)], sb).wait()
            pltpu.make_async_copy(ent_h.at[fl_ae.at[pl.ds(ob, CHN)]],
                                  ent_nb2.at[pl.ds(ob, CHN)], sc2).wait()

        def do_compute(coff, par):
            rof = par * CH
            fof = par * CHN

            def group_body(g, carry2):
                eb = g * L + iota
                rows = eb + rof
                frows = [eb * NN + fof + _splat(j) for j in range(NN)]
                relids = [plsc.load_gather(fl_ar, [frows[j]])
                          for j in range(NN)]
                s_list = [None] * NN
                for d in range(DIM):
                    ue_d = plsc.load_gather(ue_r, [rows, _splat(d)])
                    for j in range(NN):
                        term = ue_d * plsc.load_gather(
                            rel_v, [relids[j], _splat(d)])
                        s_list[j] = term if s_list[j] is None \
                            else s_list[j] + term
                m = s_list[0]
                for j in range(1, NN):
                    m = jnp.maximum(m, s_list[j])
                e_list = [jnp.exp(sj - m) for sj in s_list]
                inv = 1.0 / _tree_sum(list(e_list))
                p_list = [ej * inv for ej in e_list]
                y = [None] * DIM
                for d in range(DIM):
                    gs = [p_list[j] * plsc.load_gather(
                        ent_nb2, [frows[j], _splat(d)]) for j in range(NN)]
                    x_d = _tree_sum(gs) + plsc.load_gather(
                        io_r, [rows, _splat(d)])
                    for dp in range(DIM):
                        w_sc = plsc.load_gather(w_v, [_splat(d), _splat(dp)])
                        term = x_d * w_sc
                        y[dp] = term if y[dp] is None else y[dp] + term
                sc = None
                for dp in range(DIM):
                    yv = y[dp] + plsc.load_gather(b_v, [_splat(dp)])
                    tt = jnp.exp(jnp.abs(yv) * (-2.0))
                    r = (1.0 - tt) / (1.0 + tt)
                    t_d = jnp.where(yv < 0.0, -r, r)
                    lu = _tree_sum([plsc.load_gather(
                        usr_nb, [frows[j], _splat(dp)]) for j in range(NN)])
                    li = _tree_sum([plsc.load_gather(
                        ent_nb1, [frows[j], _splat(dp)]) for j in range(NN)])
                    ue_d = plsc.load_gather(ue_r, [rows, _splat(dp)])
                    uf = 0.0625 * lu + 0.5 * ue_d
                    if_ = 0.0625 * li + 0.5 * t_d
                    term = uf * if_
                    sc = term if sc is None else sc + term
                sig = 1.0 / (1.0 + jnp.exp(-sc))
                out_v[pl.ds(coff + g * L, L)] = sig
                return carry2

            lax.fori_loop(0, GPC, group_body, 0)

        issue_nb(0)
        issue_ueio(0, 0)
        wait_nb()
        flatten(0)
        issue_l2(0)
        issue_nb(1)
        issue_ueio(1, 1)

        def super_body(k, carry):
            a = 2 * k
            b = a + 1
            first = k < (NCHUNK // 2 - 1)
            wait_nb()
            flatten(1)
            issue_l2(1)

            @pl.when(first)
            def _():
                issue_nb(a + 2)

            wait_l2(0)
            wait_ueio(0)
            do_compute(a * CH, 0)

            @pl.when(first)
            def _():
                issue_ueio(a + 2, 0)

            @pl.when(first)
            def _():
                wait_nb()
                flatten(0)
                issue_l2(0)
                issue_nb(b + 2)

            wait_l2(1)
            wait_ueio(1)
            do_compute(b * CH, 1)

            @pl.when(first)
            def _():
                issue_ueio(b + 2, 1)

            return carry

        lax.fori_loop(0, NCHUNK // 2, super_body, 0)
        pltpu.sync_copy(out_v, out_h.at[pl.ds(base, BW)])

    return klgcn


def kernel(usr, ent, rel, W, b, u2i, i2u, adj_ent, adj_rel, u, v):
    B = u.shape[0]
    usr_rm, ent_rm, u2i_rm, i2u_rm, aepk_rm = _tc_relayout_all(
        usr.T, ent.T, u2i.T, i2u.T, adj_ent.T, adj_rel.T)
    info = plsc.get_sparse_core_info()
    fn = _build(B, info.num_cores, info.num_subcores)
    return fn(usr_rm, ent_rm, rel, W, b, u2i_rm, i2u_rm, aepk_rm,
              u.astype(jnp.int32), v.astype(jnp.int32))
